# Initial kernel scaffold; baseline (speedup 1.0000x reference)
import jax
import jax.numpy as jnp
from jax.experimental import pallas as pl


def kernel(x, edge_index, W1_self, W1_neigh, b1, W2_self, W2_neigh, b2):
    raise NotImplementedError("write your pallas kernel here")


# speedup vs baseline: 4.2518x; 4.2518x over previous
import functools

import jax
import jax.numpy as jnp
from jax import lax
from jax.experimental import pallas as pl
from jax.experimental.pallas import tpu as pltpu
from jax.experimental.pallas import tpu_sc as plsc

N = 10000
D = 128
H = D // 2
E = 320000

NC = 2
NS = 16
DW = 8

CHUNK = 128
C = -(-E // (NS * CHUNK))
E_PAD = NS * CHUNK * C
CH0 = (C + 1) // 2
ACC_N = 10240
ROWS_PT = ACC_N // NS

_sc_mesh = plsc.VectorSubcoreMesh(core_axis_name="c", subcore_axis_name="s")


def _sc_agg_body(h0, h1, srcm, dstm, zrows, zdeg, ones, part, deg,
                 sidx, didx, rows_v, ones_v, acc_s, deg_s, semg, with_deg):
    cid = lax.axis_index("c")
    sid = lax.axis_index("s")
    lo = sid * ROWS_PT
    pltpu.sync_copy(zrows.at[pl.ds(lo, ROWS_PT)], acc_s.at[pl.ds(lo, ROWS_PT)])
    if with_deg:
        pltpu.sync_copy(zdeg.at[pl.ds(lo, ROWS_PT)], deg_s.at[pl.ds(lo, ROWS_PT)])
        pltpu.sync_copy(ones, ones_v)
    plsc.subcore_barrier()

    dlo = jnp.where(cid == 0, 0, CH0)
    dhi = jnp.where(cid == 0, CH0, C)

    def step(j, carry):
        pltpu.sync_copy(srcm.at[sid, j], sidx.at[0])
        pltpu.sync_copy(dstm.at[sid, j], didx.at[0])

        @pl.when(cid == 0)
        def _():
            pltpu.async_copy(h0.at[sidx.at[0]], rows_v, semg).wait()

        @pl.when(cid == 1)
        def _():
            pltpu.async_copy(h1.at[sidx.at[0]], rows_v, semg).wait()

        pltpu.sync_copy(rows_v, acc_s.at[didx.at[0]], add=True)
        if with_deg:
            @pl.when((j >= dlo) & (j < dhi))
            def _():
                pltpu.sync_copy(ones_v, deg_s.at[didx.at[0]], add=True)
        return carry

    lax.fori_loop(0, C, step, 0)
    plsc.subcore_barrier()
    pltpu.sync_copy(acc_s.at[pl.ds(lo, ROWS_PT)], part.at[cid, pl.ds(lo, ROWS_PT)])
    if with_deg:
        pltpu.sync_copy(deg_s.at[pl.ds(lo, ROWS_PT)], deg.at[cid, pl.ds(lo, ROWS_PT)])


def _make_sc_agg(with_deg):
    if with_deg:
        body = functools.partial(_sc_agg_body, with_deg=True)
        out_type = (jax.ShapeDtypeStruct((NC, ACC_N, H), jnp.float32),
                    jax.ShapeDtypeStruct((NC, ACC_N, DW), jnp.float32))
        scratch = (
            pltpu.VMEM((2, CHUNK), jnp.int32),
            pltpu.VMEM((2, CHUNK), jnp.int32),
            pltpu.VMEM((CHUNK, H), jnp.float32),
            pltpu.VMEM((CHUNK, DW), jnp.float32),
            pltpu.VMEM_SHARED((ACC_N, H), jnp.float32),
            pltpu.VMEM_SHARED((ACC_N, DW), jnp.float32),
            pltpu.SemaphoreType.DMA,
        )
        return pl.kernel(body, out_type=out_type, mesh=_sc_mesh,
                         scratch_types=scratch,
                         compiler_params=pltpu.CompilerParams(
                             use_tc_tiling_on_sc=False))

    def body(h0, h1, srcm, dstm, zrows, part, sidx, didx, rows_v, acc_s, semg):
        _sc_agg_body(h0, h1, srcm, dstm, zrows, None, None, part, None,
                     sidx, didx, rows_v, None, acc_s, None, semg,
                     with_deg=False)

    out_type = jax.ShapeDtypeStruct((NC, ACC_N, H), jnp.float32)
    scratch = (
        pltpu.VMEM((2, CHUNK), jnp.int32),
        pltpu.VMEM((2, CHUNK), jnp.int32),
        pltpu.VMEM((CHUNK, H), jnp.float32),
        pltpu.VMEM_SHARED((ACC_N, H), jnp.float32),
        pltpu.SemaphoreType.DMA,
    )
    return pl.kernel(body, out_type=out_type, mesh=_sc_mesh,
                     scratch_types=scratch,
                     compiler_params=pltpu.CompilerParams(
                         use_tc_tiling_on_sc=False))


_sc_agg_deg = _make_sc_agg(True)
_sc_agg = _make_sc_agg(False)


def _tc_layer_body(xh_ref, p_ref, dg_ref, ws_ref, wn_ref, b_ref, o_ref, *,
                   relu, halves_out):
    dsum = dg_ref[0] + dg_ref[1]
    inv = 1.0 / jnp.maximum(dsum[:, 0:1], 1.0)
    dot = functools.partial(jnp.dot, preferred_element_type=jnp.float32)
    out = (dot(xh_ref[0], ws_ref[0]) + dot(xh_ref[1], ws_ref[1])
           + dot(p_ref[0] * inv, wn_ref[0]) + dot(p_ref[1] * inv, wn_ref[1])
           + b_ref[...])
    if relu:
        out = jnp.maximum(out, 0.0)
    if halves_out:
        o_ref[0] = out[:, :H]
        o_ref[1] = out[:, H:]
    else:
        o_ref[...] = out


_BR = 1024


def _tc_layer(xh, part, deg, w_self, w_neigh, b, relu, halves_out):
    if halves_out:
        out_shape = jax.ShapeDtypeStruct((NC, ACC_N, H), jnp.float32)
        out_spec = pl.BlockSpec((NC, _BR, H), lambda i: (0, i, 0))
    else:
        out_shape = jax.ShapeDtypeStruct((ACC_N, D), jnp.float32)
        out_spec = pl.BlockSpec((_BR, D), lambda i: (i, 0))
    return pl.pallas_call(
        functools.partial(_tc_layer_body, relu=relu, halves_out=halves_out),
        grid=(ACC_N // _BR,),
        in_specs=[
            pl.BlockSpec((NC, _BR, H), lambda i: (0, i, 0)),
            pl.BlockSpec((NC, _BR, H), lambda i: (0, i, 0)),
            pl.BlockSpec((NC, _BR, DW), lambda i: (0, i, 0)),
            pl.BlockSpec((NC, H, D), lambda i: (0, 0, 0)),
            pl.BlockSpec((NC, H, D), lambda i: (0, 0, 0)),
            pl.BlockSpec((1, D), lambda i: (0, 0)),
        ],
        out_specs=out_spec,
        out_shape=out_shape,
    )(xh, part, deg, w_self, w_neigh, b.reshape(1, D))


def _split_w(w):
    return jnp.stack([w[:H], w[H:]])


def kernel(x, edge_index, W1_self, W1_neigh, b1, W2_self, W2_neigh, b2):
    src = edge_index[0].astype(jnp.int32)
    dst = edge_index[1].astype(jnp.int32)
    pad = E_PAD - E
    srcm = jnp.concatenate([src, jnp.zeros((pad,), jnp.int32)]).reshape(NS, C, CHUNK)
    dstm = jnp.concatenate([dst, jnp.full((pad,), N, jnp.int32)]).reshape(NS, C, CHUNK)
    zrows = jnp.zeros((ACC_N, H), jnp.float32)
    zdeg = jnp.zeros((ACC_N, DW), jnp.float32)
    ones = jnp.ones((CHUNK, DW), jnp.float32)
    x_pad = jnp.zeros((ACC_N, D), jnp.float32).at[:N].set(x)
    xh = jnp.stack([x_pad[:, :H], x_pad[:, H:]])

    part1, deg = _sc_agg_deg(xh[0], xh[1], srcm, dstm, zrows, zdeg, ones)
    hh = _tc_layer(xh, part1, deg, _split_w(W1_self), _split_w(W1_neigh), b1,
                   relu=True, halves_out=True)
    part2 = _sc_agg(hh[0], hh[1], srcm, dstm, zrows)
    out = _tc_layer(hh, part2, deg, _split_w(W2_self), _split_w(W2_neigh), b2,
                    relu=False, halves_out=False)
    return out[:N]


# speedup vs baseline: 6.6209x; 1.5572x over previous
import functools

import jax
import jax.numpy as jnp
from jax import lax
from jax.experimental import pallas as pl
from jax.experimental.pallas import tpu as pltpu
from jax.experimental.pallas import tpu_sc as plsc

N = 10000
D = 128
H = D // 2
E = 320000

NC = 2
NS = 16
DW = 8

CHUNK = 128
C = 2 * (-(-E // (NS * CHUNK * 2)))
E_PAD = NS * CHUNK * C
CH0 = C // 2
ACC_N = 10240
ROWS_PT = ACC_N // NS

_sc_mesh = plsc.VectorSubcoreMesh(core_axis_name="c", subcore_axis_name="s")
_sc_params = pltpu.CompilerParams(use_tc_tiling_on_sc=False)


def _sc_agg_body(hh, idxm, zrows, zdeg, ones, part, deg,
                 sd, rows_v, ones_v, acc_s, deg_s,
                 semg0, semg1, semi0, semi1, with_deg):
    cid = lax.axis_index("c")
    sid = lax.axis_index("s")
    lo = sid * ROWS_PT
    semg = (semg0, semg1)
    semi = (semi0, semi1)
    hcol = hh.at[cid]
    pltpu.sync_copy(zrows.at[pl.ds(lo, ROWS_PT)], acc_s.at[pl.ds(lo, ROWS_PT)])
    if with_deg:
        pltpu.sync_copy(zdeg.at[pl.ds(lo, ROWS_PT)], deg_s.at[pl.ds(lo, ROWS_PT)])
        pltpu.sync_copy(ones, ones_v)
    plsc.subcore_barrier()

    dlo = jnp.where(cid == 0, 0, CH0)
    dhi = jnp.where(cid == 0, CH0, C)

    pltpu.sync_copy(idxm.at[sid, 0], sd.at[0])
    pltpu.async_copy(hcol.at[sd.at[0, 0]], rows_v.at[0], semg[0])
    pltpu.async_copy(idxm.at[sid, 1], sd.at[1], semi[1])

    def pair(t, carry):
        for p in (0, 1):
            q = 1 - p
            j = 2 * t + p

            @pl.when(j + 1 < C)
            def _():
                pltpu.make_async_copy(idxm.at[sid, j + 1], sd.at[q], semi[q]).wait()
                pltpu.async_copy(hcol.at[sd.at[q, 0]], rows_v.at[q], semg[q])

            pltpu.make_async_copy(hcol.at[sd.at[p, 0]], rows_v.at[p], semg[p]).wait()
            pltpu.sync_copy(rows_v.at[p], acc_s.at[sd.at[p, 1]], add=True)
            if with_deg:
                @pl.when((j >= dlo) & (j < dhi))
                def _():
                    pltpu.sync_copy(ones_v, deg_s.at[sd.at[p, 1]], add=True)

            @pl.when(j + 2 < C)
            def _():
                pltpu.async_copy(idxm.at[sid, j + 2], sd.at[p], semi[p])
        return carry

    lax.fori_loop(0, C // 2, pair, 0)
    plsc.subcore_barrier()
    pltpu.sync_copy(acc_s.at[pl.ds(lo, ROWS_PT)], part.at[cid, pl.ds(lo, ROWS_PT)])
    if with_deg:
        pltpu.sync_copy(deg_s.at[pl.ds(lo, ROWS_PT)], deg.at[cid, pl.ds(lo, ROWS_PT)])


def _make_sc_agg(with_deg):
    if with_deg:
        body = functools.partial(_sc_agg_body, with_deg=True)
        out_type = (jax.ShapeDtypeStruct((NC, ACC_N, H), jnp.float32),
                    jax.ShapeDtypeStruct((NC, ACC_N, DW), jnp.float32))
        scratch = (
            pltpu.VMEM((2, 2, CHUNK), jnp.int32),
            pltpu.VMEM((2, CHUNK, H), jnp.float32),
            pltpu.VMEM((CHUNK, DW), jnp.float32),
            pltpu.VMEM_SHARED((ACC_N, H), jnp.float32),
            pltpu.VMEM_SHARED((ACC_N, DW), jnp.float32),
            pltpu.SemaphoreType.DMA,
            pltpu.SemaphoreType.DMA,
            pltpu.SemaphoreType.DMA,
            pltpu.SemaphoreType.DMA,
        )
        return pl.kernel(body, out_type=out_type, mesh=_sc_mesh,
                         scratch_types=scratch, compiler_params=_sc_params)

    def body(hh, idxm, zrows, part, sd, rows_v, acc_s, g0, g1, i0, i1):
        _sc_agg_body(hh, idxm, zrows, None, None, part, None,
                     sd, rows_v, None, acc_s, None, g0, g1, i0, i1,
                     with_deg=False)

    out_type = jax.ShapeDtypeStruct((NC, ACC_N, H), jnp.float32)
    scratch = (
        pltpu.VMEM((2, 2, CHUNK), jnp.int32),
        pltpu.VMEM((2, CHUNK, H), jnp.float32),
        pltpu.VMEM_SHARED((ACC_N, H), jnp.float32),
        pltpu.SemaphoreType.DMA,
        pltpu.SemaphoreType.DMA,
        pltpu.SemaphoreType.DMA,
        pltpu.SemaphoreType.DMA,
    )
    return pl.kernel(body, out_type=out_type, mesh=_sc_mesh,
                     scratch_types=scratch, compiler_params=_sc_params)


_sc_agg_deg = _make_sc_agg(True)
_sc_agg = _make_sc_agg(False)


def _tc_layer_body(xh_ref, p_ref, dg_ref, ws_ref, wn_ref, b_ref, o_ref, *,
                   relu, halves_out):
    dsum = dg_ref[0] + dg_ref[1]
    inv = 1.0 / jnp.maximum(dsum[:, 0:1], 1.0)
    dot = functools.partial(jnp.dot, preferred_element_type=jnp.float32)
    out = (dot(xh_ref[0], ws_ref[0]) + dot(xh_ref[1], ws_ref[1])
           + dot(p_ref[0] * inv, wn_ref[0]) + dot(p_ref[1] * inv, wn_ref[1])
           + b_ref[...])
    if relu:
        out = jnp.maximum(out, 0.0)
    if halves_out:
        o_ref[0] = out[:, :H]
        o_ref[1] = out[:, H:]
    else:
        o_ref[...] = out


_BR = 1024


def _tc_layer(xh, part, deg, w_self, w_neigh, b, relu, halves_out):
    if halves_out:
        out_shape = jax.ShapeDtypeStruct((NC, ACC_N, H), jnp.float32)
        out_spec = pl.BlockSpec((NC, _BR, H), lambda i: (0, i, 0))
    else:
        out_shape = jax.ShapeDtypeStruct((ACC_N, D), jnp.float32)
        out_spec = pl.BlockSpec((_BR, D), lambda i: (i, 0))
    return pl.pallas_call(
        functools.partial(_tc_layer_body, relu=relu, halves_out=halves_out),
        grid=(ACC_N // _BR,),
        in_specs=[
            pl.BlockSpec((NC, _BR, H), lambda i: (0, i, 0)),
            pl.BlockSpec((NC, _BR, H), lambda i: (0, i, 0)),
            pl.BlockSpec((NC, _BR, DW), lambda i: (0, i, 0)),
            pl.BlockSpec((NC, H, D), lambda i: (0, 0, 0)),
            pl.BlockSpec((NC, H, D), lambda i: (0, 0, 0)),
            pl.BlockSpec((1, D), lambda i: (0, 0)),
        ],
        out_specs=out_spec,
        out_shape=out_shape,
    )(xh, part, deg, w_self, w_neigh, b.reshape(1, D))


def _split_w(w):
    return jnp.stack([w[:H], w[H:]])


def kernel(x, edge_index, W1_self, W1_neigh, b1, W2_self, W2_neigh, b2):
    src = edge_index[0].astype(jnp.int32)
    dst = edge_index[1].astype(jnp.int32)
    pad = E_PAD - E
    srcm = jnp.concatenate([src, jnp.zeros((pad,), jnp.int32)]).reshape(NS, C, CHUNK)
    dstm = jnp.concatenate([dst, jnp.full((pad,), N, jnp.int32)]).reshape(NS, C, CHUNK)
    idxm = jnp.stack([srcm, dstm], axis=2)
    zrows = jnp.zeros((ACC_N, H), jnp.float32)
    zdeg = jnp.zeros((ACC_N, DW), jnp.float32)
    ones = jnp.ones((CHUNK, DW), jnp.float32)
    x_pad = jnp.zeros((ACC_N, D), jnp.float32).at[:N].set(x)
    xh = jnp.stack([x_pad[:, :H], x_pad[:, H:]])

    part1, deg = _sc_agg_deg(xh, idxm, zrows, zdeg, ones)
    hh = _tc_layer(xh, part1, deg, _split_w(W1_self), _split_w(W1_neigh), b1,
                   relu=True, halves_out=True)
    part2 = _sc_agg(hh, idxm, zrows)
    out = _tc_layer(hh, part2, deg, _split_w(W2_self), _split_w(W2_neigh), b2,
                    relu=False, halves_out=False)
    return out[:N]
